# Initial kernel scaffold; baseline (speedup 1.0000x reference)
import jax
import jax.numpy as jnp
from jax.experimental import pallas as pl


def kernel(node_features, edge_features, params, senders, receivers):
    raise NotImplementedError("write your pallas kernel here")


# speedup vs baseline: 2.3223x; 2.3223x over previous
import functools

import jax
import jax.numpy as jnp
from jax import lax
from jax.experimental import pallas as pl
from jax.experimental.pallas import tpu as pltpu
from jax.experimental.pallas import tpu_sc as plsc

_N_NODES = 10000
_N_EDGES = 160000
_L = 128

_NW = 32
_EPW = _N_EDGES // _NW
_CH = 40
_NB = _EPW // _CH

_EBLK = 1000
_NBLK = 1000

_NPAD = 10240


def _ln(y, g, b):
    mu = jnp.mean(y, axis=-1, keepdims=True)
    var = jnp.mean((y - mu) ** 2, axis=-1, keepdims=True)
    return (y - mu) * lax.rsqrt(var + 1e-5) * g + b


def _dot(a, b):
    return jnp.dot(a, b, preferred_element_type=jnp.float32)


def _enc_body(x, w1, b1, w2, b2, g, bb, o):
    h = jnp.maximum(_dot(x[...], w1[...]) + b1[...], 0.0)
    o[...] = _ln(_dot(h, w2[...]) + b2[...], g[...], bb[...])


def _edge_body(ns, nr, e, w1a, w1b, w1c, b1, w2, b2, g, bb, o):
    x = _dot(ns[...], w1a[...]) + _dot(nr[...], w1b[...]) + _dot(e[...], w1c[...]) + b1[...]
    h = jnp.maximum(x, 0.0)
    o[...] = _ln(_dot(h, w2[...]) + b2[...], g[...], bb[...]) + e[...]


def _node_body(nl, a0, a1, w1a, w1b, b1, w2, b2, g, bb, o):
    agg = a0[...] + a1[...]
    x = _dot(nl[...], w1a[...]) + _dot(agg, w1b[...]) + b1[...]
    h = jnp.maximum(x, 0.0)
    o[...] = _ln(_dot(h, w2[...]) + b2[...], g[...], bb[...]) + nl[...]


def _dec_body(x, w1, b1, w2, b2, o):
    h = jnp.maximum(_dot(x[...], w1[...]) + b1[...], 0.0)
    o[...] = _dot(h, w2[...]) + b2[...]


def _row(v):
    return v.reshape(1, _L)


_W = pl.BlockSpec((_L, _L), lambda i: (0, 0))
_R = pl.BlockSpec((1, _L), lambda i: (0, 0))


def _blk_spec(blk):
    return pl.BlockSpec((blk, _L), lambda i: (i, 0))


def _tc_call(body, n, blk, n_data, arrays):
    grid = n // blk
    specs = [_blk_spec(blk)] * n_data + [_W if a.shape == (_L, _L) else _R
                                         for a in arrays[n_data:]]
    return pl.pallas_call(
        body,
        grid=(grid,),
        in_specs=specs,
        out_specs=_blk_spec(blk),
        out_shape=jax.ShapeDtypeStruct((n, _L), jnp.float32),
    )(*arrays)


def _sc_mesh():
    return plsc.VectorSubcoreMesh(core_axis_name="c", subcore_axis_name="s")


def _gather_pair(table, idx_s, idx_r):
    out_t = (jax.ShapeDtypeStruct((_NW, _NB, _CH, _L), jnp.float32),) * 2

    @functools.partial(
        pl.kernel,
        mesh=_sc_mesh(),
        out_type=out_t,
        scratch_types=[
            pltpu.VMEM((_NB, _CH), jnp.int32),
            pltpu.VMEM((_NB, _CH), jnp.int32),
            pltpu.VMEM((_CH, _L), jnp.float32),
            pltpu.VMEM((_CH, _L), jnp.float32),
            pltpu.SemaphoreType.DMA,
            pltpu.SemaphoreType.DMA,
        ],
    )
    def k(tab, s_i, r_i, ns_o, nr_o, sv, rv, bs, br, sem_s, sem_r):
        wid = lax.axis_index("s") * 2 + lax.axis_index("c")
        pltpu.sync_copy(s_i.at[wid], sv)
        pltpu.sync_copy(r_i.at[wid], rv)

        def body(j, carry):
            cs = pltpu.async_copy(tab.at[sv.at[j]], bs, sem_s)
            cr = pltpu.async_copy(tab.at[rv.at[j]], br, sem_r)
            cs.wait()
            pltpu.sync_copy(bs, ns_o.at[wid, j])
            cr.wait()
            pltpu.sync_copy(br, nr_o.at[wid, j])
            return carry

        lax.fori_loop(0, _NB, body, 0)

    return k(table, idx_s, idx_r)


def _segment_sum(edges4, idx_r, zeros):
    rows = _NPAD // 16

    @functools.partial(
        pl.kernel,
        mesh=_sc_mesh(),
        out_type=jax.ShapeDtypeStruct((2, _NPAD, _L), jnp.float32),
        scratch_types=[
            pltpu.VMEM((_NB, _CH), jnp.int32),
            pltpu.VMEM((_CH, _L), jnp.float32),
            pltpu.VMEM_SHARED((_NPAD, _L), jnp.float32),
            pltpu.SemaphoreType.DMA,
        ],
    )
    def k(e4, r_i, z, out, rv, buf, acc, sem):
        cid = lax.axis_index("c")
        sid = lax.axis_index("s")
        wid = sid * 2 + cid
        pltpu.sync_copy(z.at[pl.ds(sid * rows, rows)], acc.at[pl.ds(sid * rows, rows)])
        pltpu.sync_copy(r_i.at[wid], rv)
        plsc.subcore_barrier()

        def body(j, carry):
            pltpu.async_copy(e4.at[wid, j], buf, sem).wait()
            pltpu.sync_copy(buf, acc.at[rv.at[j]], add=True)
            return carry

        lax.fori_loop(0, _NB, body, 0)
        plsc.subcore_barrier()
        pltpu.sync_copy(acc.at[pl.ds(sid * rows, rows)], out.at[cid, pl.ds(sid * rows, rows)])

    return k(edges4, idx_r, zeros)


def kernel(node_features, edge_features, params, senders, receivers):
    s3 = senders.astype(jnp.int32).reshape(_NW, _NB, _CH)
    r3 = receivers.astype(jnp.int32).reshape(_NW, _NB, _CH)
    zeros = jnp.zeros((_NPAD, _L), jnp.float32)

    nf = jnp.pad(node_features, ((0, 0), (0, _L - node_features.shape[1])))
    ef = jnp.pad(edge_features, ((0, 0), (0, _L - edge_features.shape[1])))

    def enc(x, p, blk):
        l0, l1 = p["layers"]
        w1 = jnp.pad(l0["w"], ((0, _L - l0["w"].shape[0]), (0, 0)))
        arrays = [x, w1, _row(l0["b"]), l1["w"], _row(l1["b"]),
                  _row(p["ln_g"]), _row(p["ln_b"])]
        return _tc_call(_enc_body, x.shape[0], blk, 1, arrays)

    nl = enc(nf, params["node_encoder"], _NBLK)
    el = enc(ef, params["edge_encoder"], _EBLK)

    for blk_p in params["blocks"]:
        ns4, nr4 = _gather_pair(nl, s3, r3)
        ns = ns4.reshape(_N_EDGES, _L)
        nr = nr4.reshape(_N_EDGES, _L)

        ep = blk_p["edge_mlp"]
        w1 = ep["layers"][0]["w"]
        arrays = [ns, nr, el, w1[:_L], w1[_L:2 * _L], w1[2 * _L:],
                  _row(ep["layers"][0]["b"]), ep["layers"][1]["w"],
                  _row(ep["layers"][1]["b"]), _row(ep["ln_g"]), _row(ep["ln_b"])]
        new_e = _tc_call(_edge_body, _N_EDGES, _EBLK, 3, arrays)

        parts = _segment_sum(new_e.reshape(_NW, _NB, _CH, _L), r3, zeros)

        np_ = blk_p["node_mlp"]
        v1 = np_["layers"][0]["w"]
        arrays = [nl, parts[0], parts[1], v1[:_L], v1[_L:],
                  _row(np_["layers"][0]["b"]), np_["layers"][1]["w"],
                  _row(np_["layers"][1]["b"]), _row(np_["ln_g"]), _row(np_["ln_b"])]
        nl = _tc_call(_node_body, _N_NODES, _NBLK, 3, arrays)
        el = new_e

    dp = params["decoder"]
    d0, d1 = dp["layers"]
    w2 = jnp.pad(d1["w"], ((0, 0), (0, _L - d1["w"].shape[1])))
    b2 = jnp.pad(d1["b"], (0, _L - d1["b"].shape[0]))
    arrays = [nl, d0["w"], _row(d0["b"]), w2, _row(b2)]
    out = _tc_call(_dec_body, _N_NODES, _NBLK, 1, arrays)
    return out[:, :d1["b"].shape[0]]


# speedup vs baseline: 3.0636x; 1.3192x over previous
import functools

import jax
import jax.numpy as jnp
from jax import lax
from jax.experimental import pallas as pl
from jax.experimental.pallas import tpu as pltpu
from jax.experimental.pallas import tpu_sc as plsc

_N_NODES = 10000
_N_EDGES = 160000
_L = 128

_NW = 32
_EPW = _N_EDGES // _NW
_CH = 40
_NB = _EPW // _CH

_EBLK = 1000
_NBLK = 1000

_NPAD = 10240


def _ln(y, g, b):
    mu = jnp.mean(y, axis=-1, keepdims=True)
    var = jnp.mean((y - mu) ** 2, axis=-1, keepdims=True)
    return (y - mu) * lax.rsqrt(var + 1e-5) * g + b


def _dot(a, b):
    return jnp.dot(a, b, preferred_element_type=jnp.float32)


def _enc_body(x, w1, b1, w2, b2, g, bb, o):
    h = jnp.maximum(_dot(x[...], w1[...]) + b1[...], 0.0)
    o[...] = _ln(_dot(h, w2[...]) + b2[...], g[...], bb[...])


def _edge_body(ns, nr, e, w1a, w1b, w1c, b1, w2, b2, g, bb, o):
    x = _dot(ns[...], w1a[...]) + _dot(nr[...], w1b[...]) + _dot(e[...], w1c[...]) + b1[...]
    h = jnp.maximum(x, 0.0)
    o[...] = _ln(_dot(h, w2[...]) + b2[...], g[...], bb[...]) + e[...]


def _node_body(nl, a0, a1, w1a, w1b, b1, w2, b2, g, bb, o):
    agg = a0[...] + a1[...]
    x = _dot(nl[...], w1a[...]) + _dot(agg, w1b[...]) + b1[...]
    h = jnp.maximum(x, 0.0)
    o[...] = _ln(_dot(h, w2[...]) + b2[...], g[...], bb[...]) + nl[...]


def _dec_body(x, w1, b1, w2, b2, o):
    h = jnp.maximum(_dot(x[...], w1[...]) + b1[...], 0.0)
    o[...] = _dot(h, w2[...]) + b2[...]


def _row(v):
    return v.reshape(1, _L)


_W = pl.BlockSpec((_L, _L), lambda i: (0, 0))
_R = pl.BlockSpec((1, _L), lambda i: (0, 0))


def _blk_spec(blk):
    return pl.BlockSpec((blk, _L), lambda i: (i, 0))


def _tc_call(body, n, blk, n_data, arrays):
    grid = n // blk
    specs = [_blk_spec(blk)] * n_data + [_W if a.shape == (_L, _L) else _R
                                         for a in arrays[n_data:]]
    return pl.pallas_call(
        body,
        grid=(grid,),
        in_specs=specs,
        out_specs=_blk_spec(blk),
        out_shape=jax.ShapeDtypeStruct((n, _L), jnp.float32),
    )(*arrays)


def _sc_mesh():
    return plsc.VectorSubcoreMesh(core_axis_name="c", subcore_axis_name="s")


_GRP = 5
_NGRP = _NB // _GRP


def _gather_pair(table, idx_s, idx_r):
    out_t = (jax.ShapeDtypeStruct((_NW, _NB, _CH, _L), jnp.float32),) * 2

    @functools.partial(
        pl.kernel,
        mesh=_sc_mesh(),
        out_type=out_t,
        scratch_types=[
            pltpu.VMEM((_NB, _CH), jnp.int32),
            pltpu.VMEM((_NB, _CH), jnp.int32),
            pltpu.VMEM((_GRP, _CH, _L), jnp.float32),
            pltpu.VMEM((_GRP, _CH, _L), jnp.float32),
            pltpu.SemaphoreType.DMA,
        ] + [pltpu.SemaphoreType.DMA] * _GRP,
    )
    def k(tab, s_i, r_i, ns_o, nr_o, sv, rv, bs, br, sem_g, *sem_w):
        wid = lax.axis_index("s") * 2 + lax.axis_index("c")
        pltpu.sync_copy(s_i.at[wid], sv)
        pltpu.sync_copy(r_i.at[wid], rv)

        def body(g, carry):
            descs = []
            for b in range(_GRP):
                j = g * _GRP + b

                @pl.when(g > 0)
                def _(b=b, j=j):
                    pltpu.make_async_copy(bs.at[b], ns_o.at[wid, j - _GRP], sem_w[b]).wait()
                    pltpu.make_async_copy(br.at[b], nr_o.at[wid, j - _GRP], sem_w[b]).wait()

                descs.append(pltpu.async_copy(tab.at[sv.at[j]], bs.at[b], sem_g))
                descs.append(pltpu.async_copy(tab.at[rv.at[j]], br.at[b], sem_g))
            for d in descs:
                d.wait()
            for b in range(_GRP):
                j = g * _GRP + b
                pltpu.async_copy(bs.at[b], ns_o.at[wid, j], sem_w[b])
                pltpu.async_copy(br.at[b], nr_o.at[wid, j], sem_w[b])
            return carry

        lax.fori_loop(0, _NGRP, body, 0)
        for b in range(_GRP):
            j = (_NGRP - 1) * _GRP + b
            pltpu.make_async_copy(bs.at[b], ns_o.at[wid, j], sem_w[b]).wait()
            pltpu.make_async_copy(br.at[b], nr_o.at[wid, j], sem_w[b]).wait()

    return k(table, idx_s, idx_r)


def _segment_sum(edges4, idx_r, zeros):
    rows = _NPAD // 16

    @functools.partial(
        pl.kernel,
        mesh=_sc_mesh(),
        out_type=jax.ShapeDtypeStruct((2, _NPAD, _L), jnp.float32),
    scratch_types=[
            pltpu.VMEM((_NB, _CH), jnp.int32),
            pltpu.VMEM((2, 2, _CH, _L), jnp.float32),
            pltpu.VMEM_SHARED((_NPAD, _L), jnp.float32),
            pltpu.SemaphoreType.DMA,
            pltpu.SemaphoreType.DMA,
            pltpu.SemaphoreType.DMA,
        ],
    )
    def k(e4, r_i, z, out, rv, buf, acc, sem0, sem1, sem_sc):
        cid = lax.axis_index("c")
        sid = lax.axis_index("s")
        wid = sid * 2 + cid
        sems = (sem0, sem1)
        nslab = (_NB - 1) // 2
        pltpu.sync_copy(z.at[pl.ds(sid * rows, rows)], acc.at[pl.ds(sid * rows, rows)])
        pltpu.sync_copy(r_i.at[wid], rv)
        plsc.subcore_barrier()

        pltpu.async_copy(e4.at[wid, pl.ds(0, 2)], buf.at[0], sems[0])

        def process(h, gg):
            pltpu.make_async_copy(e4.at[wid, pl.ds(gg * 2, 2)],
                                  buf.at[h], sems[h]).wait()
            descs = []
            for b in range(2):
                descs.append(pltpu.async_copy(
                    buf.at[h, b], acc.at[rv.at[gg * 2 + b]], sem_sc, add=True))
            for d in descs:
                d.wait()

        def body(g, carry):
            for h in range(2):
                gg = g * 2 + h

                @pl.when(gg + 1 < nslab)
                def _(h=h, gg=gg):
                    pltpu.async_copy(e4.at[wid, pl.ds((gg + 1) * 2, 2)],
                                     buf.at[1 - h], sems[1 - h])

                process(h, gg)
            return carry

        lax.fori_loop(0, nslab // 2, body, 0)
        pltpu.sync_copy(e4.at[wid, _NB - 1], buf.at[0, 0])
        pltpu.sync_copy(buf.at[0, 0], acc.at[rv.at[_NB - 1]], add=True)
        plsc.subcore_barrier()
        pltpu.sync_copy(acc.at[pl.ds(sid * rows, rows)], out.at[cid, pl.ds(sid * rows, rows)])

    return k(edges4, idx_r, zeros)


def kernel(node_features, edge_features, params, senders, receivers):
    s3 = senders.astype(jnp.int32).reshape(_NW, _NB, _CH)
    r3 = receivers.astype(jnp.int32).reshape(_NW, _NB, _CH)
    zeros = jnp.zeros((_NPAD, _L), jnp.float32)

    nf = jnp.pad(node_features, ((0, 0), (0, _L - node_features.shape[1])))
    ef = jnp.pad(edge_features, ((0, 0), (0, _L - edge_features.shape[1])))

    def enc(x, p, blk):
        l0, l1 = p["layers"]
        w1 = jnp.pad(l0["w"], ((0, _L - l0["w"].shape[0]), (0, 0)))
        arrays = [x, w1, _row(l0["b"]), l1["w"], _row(l1["b"]),
                  _row(p["ln_g"]), _row(p["ln_b"])]
        return _tc_call(_enc_body, x.shape[0], blk, 1, arrays)

    nl = enc(nf, params["node_encoder"], _NBLK)
    el = enc(ef, params["edge_encoder"], _EBLK)

    for blk_p in params["blocks"]:
        ns4, nr4 = _gather_pair(nl, s3, r3)
        ns = ns4.reshape(_N_EDGES, _L)
        nr = nr4.reshape(_N_EDGES, _L)

        ep = blk_p["edge_mlp"]
        w1 = ep["layers"][0]["w"]
        arrays = [ns, nr, el, w1[:_L], w1[_L:2 * _L], w1[2 * _L:],
                  _row(ep["layers"][0]["b"]), ep["layers"][1]["w"],
                  _row(ep["layers"][1]["b"]), _row(ep["ln_g"]), _row(ep["ln_b"])]
        new_e = _tc_call(_edge_body, _N_EDGES, _EBLK, 3, arrays)

        parts = _segment_sum(new_e.reshape(_NW, _NB, _CH, _L), r3, zeros)

        np_ = blk_p["node_mlp"]
        v1 = np_["layers"][0]["w"]
        arrays = [nl, parts[0], parts[1], v1[:_L], v1[_L:],
                  _row(np_["layers"][0]["b"]), np_["layers"][1]["w"],
                  _row(np_["layers"][1]["b"]), _row(np_["ln_g"]), _row(np_["ln_b"])]
        nl = _tc_call(_node_body, _N_NODES, _NBLK, 3, arrays)
        el = new_e

    dp = params["decoder"]
    d0, d1 = dp["layers"]
    w2 = jnp.pad(d1["w"], ((0, 0), (0, _L - d1["w"].shape[1])))
    b2 = jnp.pad(d1["b"], (0, _L - d1["b"].shape[0]))
    arrays = [nl, d0["w"], _row(d0["b"]), w2, _row(b2)]
    out = _tc_call(_dec_body, _N_NODES, _NBLK, 1, arrays)
    return out[:, :d1["b"].shape[0]]


# speedup vs baseline: 3.4743x; 1.1341x over previous
import functools

import jax
import jax.numpy as jnp
from jax import lax
from jax.experimental import pallas as pl
from jax.experimental.pallas import tpu as pltpu
from jax.experimental.pallas import tpu_sc as plsc

_N_NODES = 10000
_N_EDGES = 160000
_L = 128

_NW = 32
_EPW = _N_EDGES // _NW
_CH = 40
_NB = _EPW // _CH

_EBLK = 1000
_NBLK = 1000

_NPAD = 10240


def _ln(y, g, b):
    mu = jnp.mean(y, axis=-1, keepdims=True)
    var = jnp.mean((y - mu) ** 2, axis=-1, keepdims=True)
    return (y - mu) * lax.rsqrt(var + 1e-5) * g + b


def _dot(a, b):
    return jnp.dot(a, b, preferred_element_type=jnp.float32)


def _enc_body(x, w1, b1, w2, b2, g, bb, o):
    h = jnp.maximum(_dot(x[...], w1[...]) + b1[...], 0.0)
    o[...] = _ln(_dot(h, w2[...]) + b2[...], g[...], bb[...])


def _edge_body(ns, nr, e, w1a, w1b, w1c, b1, w2, b2, g, bb, o):
    x = _dot(ns[...], w1a[...]) + _dot(nr[...], w1b[...]) + _dot(e[...], w1c[...]) + b1[...]
    h = jnp.maximum(x, 0.0)
    o[...] = _ln(_dot(h, w2[...]) + b2[...], g[...], bb[...]) + e[...]


def _node_body(nl, a0, a1, w1a, w1b, b1, w2, b2, g, bb, o):
    agg = a0[...] + a1[...]
    x = _dot(nl[...], w1a[...]) + _dot(agg, w1b[...]) + b1[...]
    h = jnp.maximum(x, 0.0)
    o[...] = _ln(_dot(h, w2[...]) + b2[...], g[...], bb[...]) + nl[...]


def _dec_body(x, w1, b1, w2, b2, o):
    h = jnp.maximum(_dot(x[...], w1[...]) + b1[...], 0.0)
    o[...] = _dot(h, w2[...]) + b2[...]


def _row(v):
    return v.reshape(1, _L)


_W = pl.BlockSpec((_L, _L), lambda i: (0, 0))
_R = pl.BlockSpec((1, _L), lambda i: (0, 0))


def _blk_spec(blk):
    return pl.BlockSpec((blk, _L), lambda i: (i, 0))


def _tc_call(body, n, blk, n_data, arrays):
    grid = n // blk
    specs = [_blk_spec(blk)] * n_data + [_W if a.shape == (_L, _L) else _R
                                         for a in arrays[n_data:]]
    return pl.pallas_call(
        body,
        grid=(grid,),
        in_specs=specs,
        out_specs=_blk_spec(blk),
        out_shape=jax.ShapeDtypeStruct((n, _L), jnp.float32),
    )(*arrays)


def _sc_mesh():
    return plsc.VectorSubcoreMesh(core_axis_name="c", subcore_axis_name="s")


_GRP = 5
_NGRP = _NB // _GRP
_SEG = 25
_NSEG = _NB // _SEG


def _gather_pair(table, idx_s, idx_r):
    out_t = (jax.ShapeDtypeStruct((_NW, _NB, _CH, _L), jnp.float32),) * 2

    @functools.partial(
        pl.kernel,
        mesh=_sc_mesh(),
        out_type=out_t,
        scratch_types=[
            pltpu.VMEM((2, _SEG, _CH), jnp.int32),
            pltpu.VMEM((2, _SEG, _CH), jnp.int32),
            pltpu.VMEM((2, _CH, _L), jnp.float32),
            pltpu.VMEM((2, _CH, _L), jnp.float32),
            pltpu.VMEM_SHARED((_N_NODES, _L), jnp.float32),
            pltpu.SemaphoreType.DMA,
            pltpu.SemaphoreType.DMA,
            pltpu.SemaphoreType.DMA,
            pltpu.SemaphoreType.DMA,
        ],
    )
    def k(tab, s_i, r_i, ns_o, nr_o, sv, rv, bs, br, tab_sh, sg0, sg1, sw0, sw1):
        wid = lax.axis_index("s") * 2 + lax.axis_index("c")
        sid = lax.axis_index("s")
        sem_g = (sg0, sg1)
        sem_w = (sw0, sw1)
        @pl.when(sid < 10)
        def _():
            sl = pl.ds(sid * 1000, 1000)
            pltpu.sync_copy(tab.at[sl], tab_sh.at[sl])

        pltpu.sync_copy(s_i.at[wid, 0], sv.at[0])
        pltpu.sync_copy(r_i.at[wid, 0], rv.at[0])
        plsc.subcore_barrier()

        def refill(seg):
            pltpu.sync_copy(s_i.at[wid, seg], sv.at[seg % 2])
            pltpu.sync_copy(r_i.at[wid, seg], rv.at[seg % 2])

        def idx_row(ref, j):
            return ref.at[(j // _SEG) % 2, j % _SEG]

        def issue_gathers(j, h):
            pltpu.async_copy(tab_sh.at[idx_row(sv, j)], bs.at[h], sem_g[h])
            pltpu.async_copy(tab_sh.at[idx_row(rv, j)], br.at[h], sem_g[h])

        def drain_gathers(j, h):
            pltpu.make_async_copy(tab_sh.at[idx_row(sv, j)], bs.at[h], sem_g[h]).wait()
            pltpu.make_async_copy(tab_sh.at[idx_row(rv, j)], br.at[h], sem_g[h]).wait()

        def issue_writes(j, h):
            pltpu.async_copy(bs.at[h], ns_o.at[wid, j], sem_w[h])
            pltpu.async_copy(br.at[h], nr_o.at[wid, j], sem_w[h])

        def drain_writes(j, h):
            pltpu.make_async_copy(bs.at[h], ns_o.at[wid, j], sem_w[h]).wait()
            pltpu.make_async_copy(br.at[h], nr_o.at[wid, j], sem_w[h]).wait()

        issue_gathers(0, 0)

        def body(g, carry):
            for h in range(2):
                j = g * 2 + h

                @pl.when(j > 0)
                def _(j=j, h=h):
                    drain_writes(j - 1, 1 - h)

                @pl.when(jnp.logical_and(j + 1 < _NB, (j + 1) % _SEG == 0))
                def _(j=j):
                    refill((j + 1) // _SEG)

                @pl.when(j + 1 < _NB)
                def _(j=j, h=h):
                    issue_gathers(j + 1, 1 - h)

                drain_gathers(j, h)
                issue_writes(j, h)
            return carry

        lax.fori_loop(0, _NB // 2, body, 0)
        drain_writes(_NB - 2, 1)
        drain_gathers(_NB - 1, 0)
        issue_writes(_NB - 1, 0)
        drain_writes(_NB - 1, 0)

    return k(table, idx_s, idx_r)


def _segment_sum(edges4, idx_r, zeros):
    rows = _NPAD // 16

    @functools.partial(
        pl.kernel,
        mesh=_sc_mesh(),
        out_type=jax.ShapeDtypeStruct((2, _NPAD, _L), jnp.float32),
    scratch_types=[
            pltpu.VMEM((_NB, _CH), jnp.int32),
            pltpu.VMEM((2, 2, _CH, _L), jnp.float32),
            pltpu.VMEM_SHARED((_NPAD, _L), jnp.float32),
            pltpu.SemaphoreType.DMA,
            pltpu.SemaphoreType.DMA,
            pltpu.SemaphoreType.DMA,
        ],
    )
    def k(e4, r_i, z, out, rv, buf, acc, sem0, sem1, sem_sc):
        cid = lax.axis_index("c")
        sid = lax.axis_index("s")
        wid = sid * 2 + cid
        sems = (sem0, sem1)
        nslab = (_NB - 1) // 2
        pltpu.sync_copy(z.at[pl.ds(sid * rows, rows)], acc.at[pl.ds(sid * rows, rows)])
        pltpu.sync_copy(r_i.at[wid], rv)
        plsc.subcore_barrier()

        pltpu.async_copy(e4.at[wid, pl.ds(0, 2)], buf.at[0], sems[0])

        def process(h, gg):
            pltpu.make_async_copy(e4.at[wid, pl.ds(gg * 2, 2)],
                                  buf.at[h], sems[h]).wait()
            descs = []
            for b in range(2):
                descs.append(pltpu.async_copy(
                    buf.at[h, b], acc.at[rv.at[gg * 2 + b]], sem_sc, add=True))
            for d in descs:
                d.wait()

        def body(g, carry):
            for h in range(2):
                gg = g * 2 + h

                @pl.when(gg + 1 < nslab)
                def _(h=h, gg=gg):
                    pltpu.async_copy(e4.at[wid, pl.ds((gg + 1) * 2, 2)],
                                     buf.at[1 - h], sems[1 - h])

                process(h, gg)
            return carry

        lax.fori_loop(0, nslab // 2, body, 0)
        pltpu.sync_copy(e4.at[wid, _NB - 1], buf.at[0, 0])
        pltpu.sync_copy(buf.at[0, 0], acc.at[rv.at[_NB - 1]], add=True)
        plsc.subcore_barrier()
        pltpu.sync_copy(acc.at[pl.ds(sid * rows, rows)], out.at[cid, pl.ds(sid * rows, rows)])

    return k(edges4, idx_r, zeros)


def kernel(node_features, edge_features, params, senders, receivers):
    s3 = senders.astype(jnp.int32).reshape(_NW, _NB, _CH)
    r3 = receivers.astype(jnp.int32).reshape(_NW, _NB, _CH)
    s4 = s3.reshape(_NW, _NSEG, _SEG, _CH)
    r4 = r3.reshape(_NW, _NSEG, _SEG, _CH)
    zeros = jnp.zeros((_NPAD, _L), jnp.float32)

    nf = jnp.pad(node_features, ((0, 0), (0, _L - node_features.shape[1])))
    ef = jnp.pad(edge_features, ((0, 0), (0, _L - edge_features.shape[1])))

    def enc(x, p, blk):
        l0, l1 = p["layers"]
        w1 = jnp.pad(l0["w"], ((0, _L - l0["w"].shape[0]), (0, 0)))
        arrays = [x, w1, _row(l0["b"]), l1["w"], _row(l1["b"]),
                  _row(p["ln_g"]), _row(p["ln_b"])]
        return _tc_call(_enc_body, x.shape[0], blk, 1, arrays)

    nl = enc(nf, params["node_encoder"], _NBLK)
    el = enc(ef, params["edge_encoder"], _EBLK)

    for blk_p in params["blocks"]:
        ns4, nr4 = _gather_pair(nl, s4, r4)
        ns = ns4.reshape(_N_EDGES, _L)
        nr = nr4.reshape(_N_EDGES, _L)

        ep = blk_p["edge_mlp"]
        w1 = ep["layers"][0]["w"]
        arrays = [ns, nr, el, w1[:_L], w1[_L:2 * _L], w1[2 * _L:],
                  _row(ep["layers"][0]["b"]), ep["layers"][1]["w"],
                  _row(ep["layers"][1]["b"]), _row(ep["ln_g"]), _row(ep["ln_b"])]
        new_e = _tc_call(_edge_body, _N_EDGES, _EBLK, 3, arrays)

        parts = _segment_sum(new_e.reshape(_NW, _NB, _CH, _L), r3, zeros)

        np_ = blk_p["node_mlp"]
        v1 = np_["layers"][0]["w"]
        arrays = [nl, parts[0], parts[1], v1[:_L], v1[_L:],
                  _row(np_["layers"][0]["b"]), np_["layers"][1]["w"],
                  _row(np_["layers"][1]["b"]), _row(np_["ln_g"]), _row(np_["ln_b"])]
        nl = _tc_call(_node_body, _N_NODES, _NBLK, 3, arrays)
        el = new_e

    dp = params["decoder"]
    d0, d1 = dp["layers"]
    w2 = jnp.pad(d1["w"], ((0, 0), (0, _L - d1["w"].shape[1])))
    b2 = jnp.pad(d1["b"], (0, _L - d1["b"].shape[0]))
    arrays = [nl, d0["w"], _row(d0["b"]), w2, _row(b2)]
    out = _tc_call(_dec_body, _N_NODES, _NBLK, 1, arrays)
    return out[:, :d1["b"].shape[0]]
